# baseline (speedup 1.0000x reference)
import jax
import jax.numpy as jnp
import numpy as np
from jax import lax
from jax.experimental import pallas as pl
from jax.experimental.pallas import tpu as pltpu
from jax.experimental.pallas import tpu_sc as plsc

B, L, F = 16, 4096, 16
NC, NS = 2, 16
NW = NC * NS
CHUNK = (B * L) // NW
NV = CHUNK // 16
ROWS_PER_CORE = B // NC

_MAGIC = np.float32(12582912.0)
_INV_PI = np.float32(0.3183098861837907)
_C1 = np.float32(3.141584873)
_C3 = np.float32(-5.167248249)
_C5 = np.float32(2.542875767)
_C7 = np.float32(-0.5571599603)


def _sin_pi(a):
    y = a + _MAGIC
    ib = lax.bitcast_convert_type(y, jnp.int32)
    sgn = (ib & 1) << 31
    nf = y - _MAGIC
    r = a - nf
    u = r * r
    p = ((_C7 * u + _C5) * u + _C3) * u + _C1
    sv = r * p
    return lax.bitcast_convert_type(
        lax.bitcast_convert_type(sv, jnp.int32) ^ sgn, jnp.float32)


def _tanh_exp(x):
    e = jnp.exp(x + x)
    return 1.0 - 2.0 / (e + 1.0)


def _lanesum(v, lane):
    for sh in (1, 2, 4, 8):
        v = v + v[lane ^ sh]
    return v


def _sc_body(ts_hbm, om_hbm, ph_hbm, out_hbm, part_hbm,
             ts_v, w_v, om16_v, ph16_v, part_v, pa_v, pb_v, out_v):
    c = lax.axis_index("c")
    s = lax.axis_index("s")
    blk = NS * c + s
    row = blk // 2
    half = blk % 2

    pltpu.sync_copy(ts_hbm.at[row, pl.ds(half * CHUNK, CHUNK)], ts_v)
    pltpu.sync_copy(om_hbm, om16_v)
    pltpu.sync_copy(ph_hbm, ph16_v)

    zeros = jnp.zeros((16,), jnp.float32)
    o_raw = om16_v[...]
    p_raw = ph16_v[...]
    o_pi = o_raw * _INV_PI
    p_pi = p_raw * _INV_PI
    o0 = o_raw[jnp.full((16,), 0, jnp.int32)]
    p0 = p_raw[jnp.full((16,), 0, jnp.int32)]

    @plsc.parallel_loop(0, NV, 1, unroll=4, carry=(zeros, zeros))
    def pass_a(i, carry):
        d_acc, n0 = carry
        v = ts_v[pl.ds(i * 16, 16)]
        mf = jnp.where(v != 0.0, 1.0, 0.0)
        w = jnp.exp(v) * mf
        w_v[pl.ds(i * 16, 16)] = w
        return d_acc + w, n0 + w * (o0 * v + p0)

    d_acc, n0 = pass_a
    part_v[0] = n0
    part_v[F] = d_acc

    for k in range(1, F):
        ok = o_pi[jnp.full((16,), k, jnp.int32)]
        pk = p_pi[jnp.full((16,), k, jnp.int32)]

        @plsc.parallel_loop(0, NV, 1, unroll=4, carry=zeros)
        def acc_k(i, acc):
            v = ts_v[pl.ds(i * 16, 16)]
            w = w_v[pl.ds(i * 16, 16)]
            return acc + w * _sin_pi(ok * v + pk)

        part_v[k] = acc_k

    pltpu.sync_copy(part_v, part_hbm.at[blk])
    plsc.subcore_barrier()

    @pl.when(s < ROWS_PER_CORE)
    def _epilogue():
        out_row = ROWS_PER_CORE * c + s
        lane = lax.iota(jnp.int32, 16)
        pltpu.sync_copy(part_hbm.at[NS * c + 2 * s], pa_v)
        pltpu.sync_copy(part_hbm.at[NS * c + 2 * s + 1], pb_v)
        d_s = _lanesum(pa_v[F] + pb_v[F], lane)

        n_vec = jnp.zeros((16,), jnp.float32)
        for k in range(F):
            s_k = _lanesum(pa_v[k] + pb_v[k], lane)
            n_vec = n_vec + jnp.where(lane == k, s_k, 0.0)

        out_v[...] = _tanh_exp(n_vec / (d_s + 1e-8))
        pltpu.sync_copy(out_v, out_hbm.at[pl.ds(out_row * F, F)])


@jax.jit
def kernel(timestamps, omega, phi):
    mesh = plsc.VectorSubcoreMesh(core_axis_name="c", subcore_axis_name="s")
    run = pl.kernel(
        _sc_body,
        mesh=mesh,
        out_type=(
            jax.ShapeDtypeStruct((B * F,), jnp.float32),
            jax.ShapeDtypeStruct((NW, F + 1, 16), jnp.float32),
        ),
        scratch_types=[
            pltpu.VMEM((CHUNK,), jnp.float32),
            pltpu.VMEM((CHUNK,), jnp.float32),
            pltpu.VMEM((F,), jnp.float32),
            pltpu.VMEM((F,), jnp.float32),
            pltpu.VMEM((F + 1, 16), jnp.float32),
            pltpu.VMEM((F + 1, 16), jnp.float32),
            pltpu.VMEM((F + 1, 16), jnp.float32),
            pltpu.VMEM((16,), jnp.float32),
        ],
    )
    out, _ = run(timestamps, omega, phi)
    return out.reshape(B, F)
